# baseline (speedup 1.0000x reference)
import functools

import jax
import jax.numpy as jnp
from jax import lax
from jax.experimental import pallas as pl
from jax.experimental.pallas import tpu as pltpu
from jax.experimental.pallas import tpu_sc as plsc

NC, NS = 2, 16
NW = NC * NS
LANES = 16


def _sc_body(bags_per_w, lp, d, table_hbm, tags_hbm, out_hbm,
             idx_v, rows0, rows1, acc_v, sem0, sem1):
  wid = lax.axis_index("s") * NC + lax.axis_index("c")
  base = wid * bags_per_w

  pltpu.sync_copy(tags_hbm.at[pl.ds(base, bags_per_w)],
                  idx_v.at[pl.ds(0, bags_per_w)])
  zi = jnp.zeros((LANES,), jnp.int32)
  for r in (bags_per_w, bags_per_w + 1):
    for q in range(lp // LANES):
      idx_v[r, pl.ds(q * LANES, LANES)] = zi

  nq = d // LANES

  def gather(j, buf, sem):
    return pltpu.make_async_copy(table_hbm.at[idx_v.at[j]], buf, sem)

  gather(0, rows0, sem0).start()
  gather(1, rows1, sem1).start()

  def accum(bag, buf):
    def body(r, carry):
      return tuple(carry[q] + buf[r, pl.ds(q * LANES, LANES)]
                   for q in range(nq))
    z = jnp.zeros((LANES,), jnp.float32)
    acc = lax.fori_loop(0, lp, body, (z,) * nq)
    for q in range(nq):
      acc_v[bag, pl.ds(q * LANES, LANES)] = acc[q]

  def step(i, carry):
    a = 2 * i
    gather(a, rows0, sem0).wait()
    accum(a, rows0)
    gather(a + 2, rows0, sem0).start()
    gather(a + 1, rows1, sem1).wait()
    accum(a + 1, rows1)
    gather(a + 3, rows1, sem1).start()
    return carry

  lax.fori_loop(0, bags_per_w // 2, step, 0)

  gather(bags_per_w, rows0, sem0).wait()
  gather(bags_per_w + 1, rows1, sem1).wait()

  pltpu.sync_copy(acc_v, out_hbm.at[pl.ds(base, bags_per_w)])


def _sc_embedding_sum(table, tags_p):
  b, lp = tags_p.shape
  v, d = table.shape
  bags_per_w = b // NW
  mesh = plsc.VectorSubcoreMesh(core_axis_name="c", subcore_axis_name="s",
                                num_cores=NC, num_subcores=NS)
  body = functools.partial(_sc_body, bags_per_w, lp, d)
  f = pl.kernel(
      body,
      out_type=jax.ShapeDtypeStruct((b, d), jnp.float32),
      mesh=mesh,
      scratch_types=[
          pltpu.VMEM((bags_per_w + 2, lp), jnp.int32),
          pltpu.VMEM((lp, d), jnp.float32),
          pltpu.VMEM((lp, d), jnp.float32),
          pltpu.VMEM((bags_per_w, d), jnp.float32),
          pltpu.SemaphoreType.DMA,
          pltpu.SemaphoreType.DMA,
      ],
      compiler_params=pltpu.CompilerParams(use_tc_tiling_on_sc=False),
  )
  return f(table, tags_p)


def _tc_body(tags_ref, summed_ref, w1_ref, b1_ref, w2_ref, b2_ref, out_ref):
  cnt = jnp.sum((tags_ref[...] != 0).astype(jnp.float32), axis=1,
                keepdims=True)
  pooled = summed_ref[...] / jnp.maximum(cnt, 1.0)
  h = lax.dot_general(pooled, w1_ref[...], (((1,), (1,)), ((), ())),
                      preferred_element_type=jnp.float32) + b1_ref[...]
  h = jnp.maximum(h, 0.0)
  out = lax.dot_general(h, w2_ref[...], (((1,), (1,)), ((), ())),
                        preferred_element_type=jnp.float32) + b2_ref[...]
  ss = jnp.sum(out * out, axis=1, keepdims=True)
  norm = jnp.maximum(jnp.sqrt(ss), 1e-12)
  out_ref[...] = out / norm


def _tc_mlp(tags_p, summed, w1, b1, w2, b2):
  b, d = summed.shape
  blk = 1024
  grid = b // blk
  return pl.pallas_call(
      _tc_body,
      grid=(grid,),
      in_specs=[
          pl.BlockSpec((blk, tags_p.shape[1]), lambda i: (i, 0)),
          pl.BlockSpec((blk, d), lambda i: (i, 0)),
          pl.BlockSpec((d, d), lambda i: (0, 0)),
          pl.BlockSpec((1, d), lambda i: (0, 0)),
          pl.BlockSpec((d, d), lambda i: (0, 0)),
          pl.BlockSpec((1, d), lambda i: (0, 0)),
      ],
      out_specs=pl.BlockSpec((blk, d), lambda i: (i, 0)),
      out_shape=jax.ShapeDtypeStruct((b, d), jnp.float32),
  )(tags_p, summed, w1, b1, w2, b2)


def kernel(tag_ids, table, W1, b1, W2, b2):
  b, l = tag_ids.shape
  d = table.shape[1]
  lp = 64
  tags = jnp.asarray(tag_ids, jnp.int32)
  tags_p = jnp.pad(tags, ((0, 0), (0, lp - l)))
  summed = _sc_embedding_sum(table, tags_p)
  return _tc_mlp(tags_p, summed, W1, b1.reshape(1, d), W2, b2.reshape(1, d))


# speedup vs baseline: 2.7901x; 2.7901x over previous
import functools

import jax
import jax.numpy as jnp
from jax import lax
from jax.experimental import pallas as pl
from jax.experimental.pallas import tpu as pltpu
from jax.experimental.pallas import tpu_sc as plsc

NC, NS = 2, 16
NW = NC * NS
LANES = 16
DEPTH = 4
BAGS = 4


def _sc_body(chunks_per_w, l, d, table_hbm, tags_hbm, out_hbm,
             idx_v, rows_v, acc_v, *sems):
  wid = lax.axis_index("s") * NC + lax.axis_index("c")
  base = wid * chunks_per_w
  nq = d // LANES

  pltpu.sync_copy(tags_hbm.at[pl.ds(base, chunks_per_w)], idx_v)

  def gather(c, b, sem):
    return pltpu.make_async_copy(table_hbm.at[idx_v.at[c]], rows_v.at[b], sem)

  for b in range(DEPTH):
    gather(b, b, sems[b]).start()

  def accum(c, b):
    for s in range(BAGS):
      def body(r, carry):
        return tuple(carry[q] + rows_v[b, s * l + r, pl.ds(q * LANES, LANES)]
                     for q in range(nq))
      z = jnp.zeros((LANES,), jnp.float32)
      acc = lax.fori_loop(0, l, body, (z,) * nq)
      for q in range(nq):
        acc_v[c * BAGS + s, pl.ds(q * LANES, LANES)] = acc[q]

  def group(g, carry):
    first = g * DEPTH
    for b in range(DEPTH):
      gather(first + b, b, sems[b]).wait()
      accum(first + b, b)
      nxt = first + b + DEPTH

      @pl.when(nxt < chunks_per_w)
      def _():
        gather(nxt, b, sems[b]).start()
    return carry

  lax.fori_loop(0, chunks_per_w // DEPTH, group, 0)

  pltpu.sync_copy(acc_v, out_hbm.at[pl.ds(base * BAGS, chunks_per_w * BAGS)])


def _sc_embedding_sum(table, tags4, l):
  nrows, row = tags4.shape
  v, d = table.shape
  b = nrows * BAGS
  chunks_per_w = nrows // NW
  mesh = plsc.VectorSubcoreMesh(core_axis_name="c", subcore_axis_name="s",
                                num_cores=NC, num_subcores=NS)
  body = functools.partial(_sc_body, chunks_per_w, l, d)
  f = pl.kernel(
      body,
      out_type=jax.ShapeDtypeStruct((b, d), jnp.float32),
      mesh=mesh,
      scratch_types=[
          pltpu.VMEM((chunks_per_w, row), jnp.int32),
          pltpu.VMEM((DEPTH, row, d), jnp.float32),
          pltpu.VMEM((chunks_per_w * BAGS, d), jnp.float32),
      ] + [pltpu.SemaphoreType.DMA] * DEPTH,
      compiler_params=pltpu.CompilerParams(use_tc_tiling_on_sc=False),
  )
  return f(table, tags4)


def _tc_body(tags_ref, summed_ref, w1_ref, b1_ref, w2_ref, b2_ref, out_ref):
  cnt = jnp.sum((tags_ref[...] != 0).astype(jnp.float32), axis=1,
                keepdims=True)
  pooled = summed_ref[...] / jnp.maximum(cnt, 1.0)
  h = lax.dot_general(pooled, w1_ref[...], (((1,), (1,)), ((), ())),
                      preferred_element_type=jnp.float32) + b1_ref[...]
  h = jnp.maximum(h, 0.0)
  out = lax.dot_general(h, w2_ref[...], (((1,), (1,)), ((), ())),
                        preferred_element_type=jnp.float32) + b2_ref[...]
  ss = jnp.sum(out * out, axis=1, keepdims=True)
  norm = jnp.maximum(jnp.sqrt(ss), 1e-12)
  out_ref[...] = out / norm


def _tc_mlp(tags, summed, w1, b1, w2, b2):
  b, d = summed.shape
  blk = 1024
  grid = b // blk
  return pl.pallas_call(
      _tc_body,
      grid=(grid,),
      in_specs=[
          pl.BlockSpec((blk, tags.shape[1]), lambda i: (i, 0)),
          pl.BlockSpec((blk, d), lambda i: (i, 0)),
          pl.BlockSpec((d, d), lambda i: (0, 0)),
          pl.BlockSpec((1, d), lambda i: (0, 0)),
          pl.BlockSpec((d, d), lambda i: (0, 0)),
          pl.BlockSpec((1, d), lambda i: (0, 0)),
      ],
      out_specs=pl.BlockSpec((blk, d), lambda i: (i, 0)),
      out_shape=jax.ShapeDtypeStruct((b, d), jnp.float32),
  )(tags, summed, w1, b1, w2, b2)


def kernel(tag_ids, table, W1, b1, W2, b2):
  b, l = tag_ids.shape
  d = table.shape[1]
  tags = jnp.asarray(tag_ids, jnp.int32)
  tags4 = tags.reshape(b // BAGS, BAGS * l)
  summed = _sc_embedding_sum(table, tags4, l)
  return _tc_mlp(tags, summed, W1, b1.reshape(1, d), W2, b2.reshape(1, d))
